# Initial kernel scaffold; baseline (speedup 1.0000x reference)
import jax
import jax.numpy as jnp
from jax.experimental import pallas as pl


def kernel(h, W):
    raise NotImplementedError("write your pallas kernel here")


# speedup vs baseline: 6.0457x; 6.0457x over previous
import functools

import jax
import jax.numpy as jnp
from jax import lax
from jax.experimental import pallas as pl
from jax.experimental.pallas import tpu as pltpu
from jax.experimental.pallas import tpu_sc as plsc

_LANES = 16
_NC = 2
_NS = 16
_NW = _NC * _NS
_BN = 512


def _matmul_t_kernel(w_ref, h_ref, out_ref):
    out_ref[...] = lax.dot_general(
        w_ref[...], h_ref[...],
        dimension_numbers=(((1,), (1,)), ((), ())),
        preferred_element_type=jnp.float32,
    )


def _logits_t(h, W):
    n, d = h.shape
    e = W.shape[0]
    grid = (n // _BN,)
    return pl.pallas_call(
        _matmul_t_kernel,
        grid=grid,
        in_specs=[
            pl.BlockSpec((e, d), lambda i: (0, 0)),
            pl.BlockSpec((_BN, d), lambda i: (i, 0)),
        ],
        out_specs=pl.BlockSpec((e, _BN), lambda i: (0, i)),
        out_shape=jax.ShapeDtypeStruct((e, n), jnp.float32),
        compiler_params=pltpu.CompilerParams(
            dimension_semantics=("arbitrary",),
        ),
    )(W, h)


def _topk_body(lt_ref, w1_out, w2_out, i1_out, i2_out,
               slab, w1b, w2b, i1b, i2b, *, n_experts, rpw):
    cid = lax.axis_index("c")
    sid = lax.axis_index("s")
    wid = sid * _NC + cid
    base = wid * rpw
    pltpu.sync_copy(lt_ref.at[:, pl.ds(base, rpw)], slab)

    def group(g, carry):
        r0 = g * _LANES
        m1 = jnp.full((_LANES,), -jnp.inf, jnp.float32)
        m2 = jnp.full((_LANES,), -jnp.inf, jnp.float32)
        i1 = jnp.zeros((_LANES,), jnp.int32)
        i2 = jnp.zeros((_LANES,), jnp.int32)
        for ex in range(n_experts):
            lv = slab[ex, pl.ds(r0, _LANES)]
            gt1 = lv > m1
            gt2 = lv > m2
            m2 = jnp.where(gt1, m1, jnp.where(gt2, lv, m2))
            i2 = jnp.where(gt1, i1, jnp.where(gt2, jnp.int32(ex), i2))
            m1 = jnp.where(gt1, lv, m1)
            i1 = jnp.where(gt1, jnp.int32(ex), i1)
        ed = jnp.exp(m2 - m1)
        s = ed + jnp.float32(1.0)
        w1b[pl.ds(r0, _LANES)] = jnp.float32(1.0) / s
        w2b[pl.ds(r0, _LANES)] = ed / s
        i1b[pl.ds(r0, _LANES)] = i1
        i2b[pl.ds(r0, _LANES)] = i2
        return carry

    lax.fori_loop(0, rpw // _LANES, group, 0)
    pltpu.sync_copy(w1b, w1_out.at[pl.ds(base, rpw)])
    pltpu.sync_copy(w2b, w2_out.at[pl.ds(base, rpw)])
    pltpu.sync_copy(i1b, i1_out.at[pl.ds(base, rpw)])
    pltpu.sync_copy(i2b, i2_out.at[pl.ds(base, rpw)])


def _topk_sc(logits_t):
    e, n = logits_t.shape
    rpw = n // _NW
    mesh = plsc.VectorSubcoreMesh(
        core_axis_name="c", subcore_axis_name="s",
        num_cores=_NC, num_subcores=_NS,
    )
    body = functools.partial(_topk_body, n_experts=e, rpw=rpw)
    call = pl.kernel(
        body,
        out_type=(
            jax.ShapeDtypeStruct((n,), jnp.float32),
            jax.ShapeDtypeStruct((n,), jnp.float32),
            jax.ShapeDtypeStruct((n,), jnp.int32),
            jax.ShapeDtypeStruct((n,), jnp.int32),
        ),
        mesh=mesh,
        scratch_types=[
            pltpu.VMEM((e, rpw), jnp.float32),
            pltpu.VMEM((rpw,), jnp.float32),
            pltpu.VMEM((rpw,), jnp.float32),
            pltpu.VMEM((rpw,), jnp.int32),
            pltpu.VMEM((rpw,), jnp.int32),
        ],
    )
    return call(logits_t)


def kernel(h, W):
    lt = _logits_t(h, W)
    w1, w2, i1, i2 = _topk_sc(lt)
    return jnp.stack((w1, w2), axis=1), jnp.stack((i1, i2), axis=1)


# speedup vs baseline: 8.4185x; 1.3925x over previous
import functools

import jax
import jax.numpy as jnp
from jax import lax
from jax.experimental import pallas as pl
from jax.experimental.pallas import tpu as pltpu
from jax.experimental.pallas import tpu_sc as plsc

_LANES = 16
_NC = 2
_NS = 16
_NW = _NC * _NS
_BN = 2048


def _matmul_t_kernel(w_ref, h_ref, out_ref):
    out_ref[...] = lax.dot_general(
        w_ref[...], h_ref[...],
        dimension_numbers=(((1,), (1,)), ((), ())),
        preferred_element_type=jnp.float32,
    )


def _logits_t(h, W):
    n, d = h.shape
    e = W.shape[0]
    grid = (n // _BN,)
    return pl.pallas_call(
        _matmul_t_kernel,
        grid=grid,
        in_specs=[
            pl.BlockSpec((e, d), lambda i: (0, 0)),
            pl.BlockSpec((_BN, d), lambda i: (i, 0)),
        ],
        out_specs=pl.BlockSpec((e, _BN), lambda i: (0, i)),
        out_shape=jax.ShapeDtypeStruct((e, n), jnp.float32),
        compiler_params=pltpu.CompilerParams(
            dimension_semantics=("arbitrary",),
        ),
    )(W, h)


def _topk_body(lt_ref, w1_out, w2_out, i1_out, i2_out,
               slab, w1b, w2b, i1b, i2b, *, n_experts, rpw):
    cid = lax.axis_index("c")
    sid = lax.axis_index("s")
    wid = sid * _NC + cid
    base = wid * rpw
    pltpu.sync_copy(lt_ref.at[:, pl.ds(base, rpw)], slab)

    def group(g, carry):
        r0 = g * _LANES
        m1 = jnp.full((_LANES,), -jnp.inf, jnp.float32)
        m2 = jnp.full((_LANES,), -jnp.inf, jnp.float32)
        i1 = jnp.zeros((_LANES,), jnp.int32)
        i2 = jnp.zeros((_LANES,), jnp.int32)
        for ex in range(n_experts):
            lv = slab[ex, pl.ds(r0, _LANES)]
            gt1 = lv > m1
            gt2 = lv > m2
            m2 = jnp.where(gt1, m1, jnp.where(gt2, lv, m2))
            i2 = jnp.where(gt1, i1, jnp.where(gt2, jnp.int32(ex), i2))
            m1 = jnp.where(gt1, lv, m1)
            i1 = jnp.where(gt1, jnp.int32(ex), i1)
        ed = jnp.exp(m2 - m1)
        s = ed + jnp.float32(1.0)
        w1b[pl.ds(r0, _LANES)] = jnp.float32(1.0) / s
        w2b[pl.ds(r0, _LANES)] = ed / s
        i1b[pl.ds(r0, _LANES)] = i1
        i2b[pl.ds(r0, _LANES)] = i2
        return carry

    lax.fori_loop(0, rpw // _LANES, group, 0)
    pltpu.sync_copy(w1b, w1_out.at[pl.ds(base, rpw)])
    pltpu.sync_copy(w2b, w2_out.at[pl.ds(base, rpw)])
    pltpu.sync_copy(i1b, i1_out.at[pl.ds(base, rpw)])
    pltpu.sync_copy(i2b, i2_out.at[pl.ds(base, rpw)])


def _topk_sc(logits_t):
    e, n = logits_t.shape
    rpw = n // _NW
    mesh = plsc.VectorSubcoreMesh(
        core_axis_name="c", subcore_axis_name="s",
        num_cores=_NC, num_subcores=_NS,
    )
    body = functools.partial(_topk_body, n_experts=e, rpw=rpw)
    call = pl.kernel(
        body,
        out_type=(
            jax.ShapeDtypeStruct((n,), jnp.float32),
            jax.ShapeDtypeStruct((n,), jnp.float32),
            jax.ShapeDtypeStruct((n,), jnp.int32),
            jax.ShapeDtypeStruct((n,), jnp.int32),
        ),
        mesh=mesh,
        scratch_types=[
            pltpu.VMEM((e, rpw), jnp.float32),
            pltpu.VMEM((rpw,), jnp.float32),
            pltpu.VMEM((rpw,), jnp.float32),
            pltpu.VMEM((rpw,), jnp.int32),
            pltpu.VMEM((rpw,), jnp.int32),
        ],
    )
    return call(logits_t)


def kernel(h, W):
    lt = _logits_t(h, W)
    w1, w2, i1, i2 = _topk_sc(lt)
    return jnp.stack((w1, w2), axis=1), jnp.stack((i1, i2), axis=1)


# speedup vs baseline: 8.5642x; 1.0173x over previous
import functools

import jax
import jax.numpy as jnp
from jax import lax
from jax.experimental import pallas as pl
from jax.experimental.pallas import tpu as pltpu
from jax.experimental.pallas import tpu_sc as plsc

_LANES = 16
_NC = 2
_NS = 16
_NW = _NC * _NS
_BN = 4096


def _matmul_t_kernel(w_ref, h_ref, out_ref):
    out_ref[...] = lax.dot_general(
        w_ref[...], h_ref[...],
        dimension_numbers=(((1,), (1,)), ((), ())),
        preferred_element_type=jnp.float32,
    )


def _logits_t(h, W):
    n, d = h.shape
    e = W.shape[0]
    grid = (n // _BN,)
    return pl.pallas_call(
        _matmul_t_kernel,
        grid=grid,
        in_specs=[
            pl.BlockSpec((e, d), lambda i: (0, 0)),
            pl.BlockSpec((_BN, d), lambda i: (i, 0)),
        ],
        out_specs=pl.BlockSpec((e, _BN), lambda i: (0, i)),
        out_shape=jax.ShapeDtypeStruct((e, n), jnp.float32),
        compiler_params=pltpu.CompilerParams(
            dimension_semantics=("arbitrary",),
        ),
    )(W, h)


def _topk_body(lt_ref, w1_out, w2_out, i1_out, i2_out,
               slab, w1b, w2b, i1b, i2b, *, n_experts, rpw):
    cid = lax.axis_index("c")
    sid = lax.axis_index("s")
    wid = sid * _NC + cid
    base = wid * rpw
    pltpu.sync_copy(lt_ref.at[:, pl.ds(base, rpw)], slab)

    def group(g, carry):
        r0 = g * _LANES
        m1 = jnp.full((_LANES,), -jnp.inf, jnp.float32)
        m2 = jnp.full((_LANES,), -jnp.inf, jnp.float32)
        i1 = jnp.zeros((_LANES,), jnp.int32)
        i2 = jnp.zeros((_LANES,), jnp.int32)
        for ex in range(n_experts):
            lv = slab[ex, pl.ds(r0, _LANES)]
            gt1 = lv > m1
            gt2 = lv > m2
            m2 = jnp.where(gt1, m1, jnp.where(gt2, lv, m2))
            i2 = jnp.where(gt1, i1, jnp.where(gt2, jnp.int32(ex), i2))
            m1 = jnp.where(gt1, lv, m1)
            i1 = jnp.where(gt1, jnp.int32(ex), i1)
        ed = jnp.exp(m2 - m1)
        s = ed + jnp.float32(1.0)
        w1b[pl.ds(r0, _LANES)] = jnp.float32(1.0) / s
        w2b[pl.ds(r0, _LANES)] = ed / s
        i1b[pl.ds(r0, _LANES)] = i1
        i2b[pl.ds(r0, _LANES)] = i2
        return carry

    lax.fori_loop(0, rpw // _LANES, group, 0)
    pltpu.sync_copy(w1b, w1_out.at[pl.ds(base, rpw)])
    pltpu.sync_copy(w2b, w2_out.at[pl.ds(base, rpw)])
    pltpu.sync_copy(i1b, i1_out.at[pl.ds(base, rpw)])
    pltpu.sync_copy(i2b, i2_out.at[pl.ds(base, rpw)])


def _topk_sc(logits_t):
    e, n = logits_t.shape
    rpw = n // _NW
    mesh = plsc.VectorSubcoreMesh(
        core_axis_name="c", subcore_axis_name="s",
        num_cores=_NC, num_subcores=_NS,
    )
    body = functools.partial(_topk_body, n_experts=e, rpw=rpw)
    call = pl.kernel(
        body,
        out_type=(
            jax.ShapeDtypeStruct((n,), jnp.float32),
            jax.ShapeDtypeStruct((n,), jnp.float32),
            jax.ShapeDtypeStruct((n,), jnp.int32),
            jax.ShapeDtypeStruct((n,), jnp.int32),
        ),
        mesh=mesh,
        scratch_types=[
            pltpu.VMEM((e, rpw), jnp.float32),
            pltpu.VMEM((rpw,), jnp.float32),
            pltpu.VMEM((rpw,), jnp.float32),
            pltpu.VMEM((rpw,), jnp.int32),
            pltpu.VMEM((rpw,), jnp.int32),
        ],
    )
    return call(logits_t)


def kernel(h, W):
    lt = _logits_t(h, W)
    w1, w2, i1, i2 = _topk_sc(lt)
    return jnp.stack((w1, w2), axis=1), jnp.stack((i1, i2), axis=1)
